# baseline (speedup 1.0000x reference)
import jax
import jax.numpy as jnp
from jax import lax
from jax.experimental import pallas as pl
from jax.experimental.pallas import tpu as pltpu
from jax.experimental.pallas import tpu_sc as plsc

ROWS = 16384
COLS = 2048
LANES_SC = 16
SUB = COLS // LANES_SC
NUM_WORKERS = 32
ROWS_PER_W = ROWS // NUM_WORKERS
CHUNK_ROWS = 8
CHUNK_ELEMS = CHUNK_ROWS * COLS
CHUNK_VREGS = CHUNK_ROWS * SUB
N_CHUNKS = ROWS_PER_W // CHUNK_ROWS


def _sc_flip(in_hbm, out_hbm, in_v, out_v):
    c = lax.axis_index("c")
    s = lax.axis_index("s")
    wid = s * 2 + c
    base = wid * (ROWS_PER_W * COLS)
    iota = lax.iota(jnp.int32, LANES_SC)

    def chunk_body(ci, carry):
        off = base + ci * CHUNK_ELEMS
        pltpu.sync_copy(in_hbm.at[pl.ds(off, CHUNK_ELEMS)], in_v)

        @plsc.parallel_loop(0, CHUNK_VREGS, unroll=8)
        def _(j):
            r = j >> 7
            k = j & (SUB - 1)
            src_base = (r << 11) + (COLS - 1) - (k << 4)
            x = plsc.load_gather(in_v, [src_base - iota])
            out_v[pl.ds(j * LANES_SC, LANES_SC)] = x

        pltpu.sync_copy(out_v, out_hbm.at[pl.ds(off, CHUNK_ELEMS)])
        return carry

    lax.fori_loop(0, N_CHUNKS, chunk_body, 0)


def kernel(inputs, cond_inputs):
    flat_in = inputs.reshape(ROWS * COLS)
    mesh = plsc.VectorSubcoreMesh(core_axis_name="c", subcore_axis_name="s")
    f = pl.kernel(
        _sc_flip,
        mesh=mesh,
        out_type=jax.ShapeDtypeStruct((ROWS * COLS,), jnp.float32),
        compiler_params=pltpu.CompilerParams(needs_layout_passes=False),
        scratch_types=[
            pltpu.VMEM((CHUNK_ELEMS,), jnp.float32),
            pltpu.VMEM((CHUNK_ELEMS,), jnp.float32),
        ],
    )
    out = f(flat_in)
    return (out.reshape(ROWS, COLS), 0.0)


# speedup vs baseline: 1.2512x; 1.2512x over previous
import jax
import jax.numpy as jnp
from jax import lax
from jax.experimental import pallas as pl
from jax.experimental.pallas import tpu as pltpu
from jax.experimental.pallas import tpu_sc as plsc

ROWS = 16384
COLS = 2048
LANES_SC = 16
SUB = COLS // LANES_SC
NUM_WORKERS = 32
ROWS_PER_W = ROWS // NUM_WORKERS
CHUNK_ROWS = 16
CHUNK_ELEMS = CHUNK_ROWS * COLS
PAIRS = CHUNK_ROWS * (SUB // 2)
N_CHUNKS = ROWS_PER_W // CHUNK_ROWS


def _sc_flip(in_hbm, out_hbm, v0, v1, sin0, sin1, sout0, sout1):
    c = lax.axis_index("c")
    s = lax.axis_index("s")
    wid = s * 2 + c
    base = wid * (ROWS_PER_W * COLS)
    bufs = (v0, v1)
    sins = (sin0, sin1)
    souts = (sout0, sout1)
    iota = lax.iota(jnp.int32, LANES_SC)

    def off(ci):
        return base + ci * CHUNK_ELEMS

    pltpu.async_copy(in_hbm.at[pl.ds(off(0), CHUNK_ELEMS)], v0, sin0)

    def outer(g, carry):
        for b in range(2):
            ci = 2 * g + b
            nb = 1 - b

            @pl.when(ci >= 1)
            def _():
                pltpu.make_async_copy(
                    bufs[nb],
                    out_hbm.at[pl.ds(off(ci - 1), CHUNK_ELEMS)],
                    souts[nb],
                ).wait()

            @pl.when(ci + 1 < N_CHUNKS)
            def _():
                pltpu.async_copy(
                    in_hbm.at[pl.ds(off(ci + 1), CHUNK_ELEMS)],
                    bufs[nb], sins[nb],
                )

            pltpu.make_async_copy(
                in_hbm.at[pl.ds(off(ci), CHUNK_ELEMS)], bufs[b], sins[b]
            ).wait()

            @plsc.parallel_loop(0, PAIRS, unroll=8)
            def _(j):
                r = j >> 6
                k = j & (SUB // 2 - 1)
                a = (r << 11) + (k << 4)
                bo = (r << 11) + ((SUB - 1 - k) << 4)
                x = plsc.load_gather(bufs[b], [(bo + LANES_SC - 1) - iota])
                y = plsc.load_gather(bufs[b], [(a + LANES_SC - 1) - iota])
                bufs[b][pl.ds(a, LANES_SC)] = x
                bufs[b][pl.ds(bo, LANES_SC)] = y

            pltpu.async_copy(
                bufs[b], out_hbm.at[pl.ds(off(ci), CHUNK_ELEMS)], souts[b]
            )
        return carry

    lax.fori_loop(0, N_CHUNKS // 2, outer, 0)

    pltpu.make_async_copy(
        bufs[1], out_hbm.at[pl.ds(off(N_CHUNKS - 1), CHUNK_ELEMS)], souts[1]
    ).wait()


def kernel(inputs, cond_inputs):
    flat_in = inputs.reshape(ROWS * COLS)
    mesh = plsc.VectorSubcoreMesh(core_axis_name="c", subcore_axis_name="s")
    f = pl.kernel(
        _sc_flip,
        mesh=mesh,
        out_type=jax.ShapeDtypeStruct((ROWS * COLS,), jnp.float32),
        compiler_params=pltpu.CompilerParams(needs_layout_passes=False),
        scratch_types=[
            pltpu.VMEM((CHUNK_ELEMS,), jnp.float32),
            pltpu.VMEM((CHUNK_ELEMS,), jnp.float32),
            pltpu.SemaphoreType.DMA,
            pltpu.SemaphoreType.DMA,
            pltpu.SemaphoreType.DMA,
            pltpu.SemaphoreType.DMA,
        ],
    )
    out = f(flat_in)
    return (out.reshape(ROWS, COLS), 0.0)
